# baseline (speedup 1.0000x reference)
import functools

import jax
import jax.numpy as jnp
from jax import lax
from jax.experimental import pallas as pl
from jax.experimental.pallas import tpu as pltpu
from jax.experimental.pallas import tpu_sc as plsc

F32 = jnp.float32
NSUB = 16
NCORE = 2
NW = NCORE * NSUB
KB = 1024
RPB = KB // 128


def _leaky(x):
    return jnp.maximum(x, 0.2 * x)


def _init_body(nf_ref, wpt_ref, wpb_ref, bp1_ref, wrt_ref, br1_ref, nl_ref,
               pwr_ref, pwrf_ref, s_ref, d_ref, c0_ref):
    x = nf_ref[...]
    bn = x.shape[0]
    z2 = jnp.zeros((bn, 2), F32)
    a0 = jnp.dot(x, wpt_ref[...], preferred_element_type=F32)
    b0 = jnp.dot(x, wpb_ref[...], preferred_element_type=F32) + bp1_ref[...]
    c0_ref[...] = jnp.dot(x, wrt_ref[...], preferred_element_type=F32) + br1_ref[...]
    em1 = jnp.where(nl_ref[...] == 1.0, 1.0, 0.0).astype(F32)
    s_ref[...] = jnp.concatenate([a0, pwr_ref[...], pwrf_ref[...], z2], axis=1)
    d_ref[...] = jnp.concatenate([b0, em1, z2, z2, z2, jnp.zeros((bn, 1), F32)],
                                 axis=1)


def _init_call(nf_p, wpt, wpb, bp1r, wrt, br1r, nl_p, pwr_p, pwrf_p, n2, bn):
    grid = (n2 // bn,)
    io = lambda i: (i, 0)
    w0 = lambda i: (0, 0)
    return pl.pallas_call(
        _init_body,
        grid=grid,
        in_specs=[
            pl.BlockSpec((bn, 128), io),
            pl.BlockSpec((128, 8), w0),
            pl.BlockSpec((128, 8), w0),
            pl.BlockSpec((1, 8), w0),
            pl.BlockSpec((128, 8), w0),
            pl.BlockSpec((1, 8), w0),
            pl.BlockSpec((bn, 1), io),
            pl.BlockSpec((bn, 2), io),
            pl.BlockSpec((bn, 4), io),
        ],
        out_specs=[
            pl.BlockSpec((bn, 16), io),
            pl.BlockSpec((bn, 16), io),
            pl.BlockSpec((bn, 8), io),
        ],
        out_shape=[
            jax.ShapeDtypeStruct((n2, 16), F32),
            jax.ShapeDtypeStruct((n2, 16), F32),
            jax.ShapeDtypeStruct((n2, 8), F32),
        ],
    )(nf_p, wpt, wpb, bp1r, wrt, br1r, nl_p, pwr_p, pwrf_p)


def _edge_body(n2, nblk, s_hbm, d_hbm, srcr, dstr, w_hbm, z_hbm, out_hbm,
               acc, wbuf, idx_s, idx_d, sbuf, dbuf, obuf, sem1, sem2):
    c = lax.axis_index("c")
    s = lax.axis_index("s")
    wid = c * NSUB + s
    rpt = n2 // NSUB
    pltpu.sync_copy(z_hbm.at[pl.ds(s * rpt, rpt)], acc.at[pl.ds(s * rpt, rpt)])
    pltpu.sync_copy(w_hbm, wbuf)
    plsc.subcore_barrier()

    row_base = wid * nblk * RPB
    iota = lax.iota(jnp.int32, 16)
    cidx = [jnp.full((16,), cc, jnp.int32) for cc in range(16)]

    def group_body(g4, car):
        rows = [g4 * 64 + 16 * t + iota for t in range(4)]
        h = [[_leaky(plsc.load_gather(sbuf, [rows[t], cidx[f]])
                     + plsc.load_gather(dbuf, [rows[t], cidx[f]]))
              for f in range(8)] for t in range(4)]
        m = [plsc.load_gather(dbuf, [rows[t], cidx[8]]) for t in range(4)]
        km = [None] * 4
        for j in range(9):
            w = [wbuf[f * 9 + j] for f in range(8)]
            b = wbuf[72 + j]
            for t in range(4):
                e = b
                for f in range(8):
                    e = e + h[t][f] * w[f]
                if j == 0:
                    km[t] = m[t] / (1.0 + jnp.exp(-e))
                else:
                    plsc.store_scatter(obuf, [rows[t], cidx[j - 1]],
                                       km[t] * e)
        for t in range(4):
            plsc.store_scatter(obuf, [rows[t], cidx[8]], m[t])
            for q in range(6):
                sv = plsc.load_gather(sbuf, [rows[t], cidx[8 + q]])
                plsc.store_scatter(obuf, [rows[t], cidx[9 + q]], m[t] * sv)
            plsc.store_scatter(obuf, [rows[t], cidx[15]],
                               jnp.zeros((16,), F32))
        return car

    def blk_body(blk, car):
        r0 = row_base + blk * RPB
        pltpu.sync_copy(srcr.at[pl.ds(r0, RPB)], idx_s)
        pltpu.sync_copy(dstr.at[pl.ds(r0, RPB)], idx_d)
        descs = []
        for q in range(RPB):
            descs.append(pltpu.async_copy(
                s_hbm.at[idx_s.at[q]], sbuf.at[pl.ds(q * 128, 128)], sem1))
            descs.append(pltpu.async_copy(
                d_hbm.at[idx_d.at[q]], dbuf.at[pl.ds(q * 128, 128)], sem2))
        for dsc in descs:
            dsc.wait()
        car = lax.fori_loop(0, KB // 64, group_body, car)
        for q in range(RPB):
            pltpu.sync_copy(obuf.at[pl.ds(q * 128, 128)],
                            acc.at[idx_d.at[q]], add=True)
        return car

    lax.fori_loop(0, nblk, blk_body, jnp.int32(0))

    plsc.subcore_barrier()
    pltpu.sync_copy(acc.at[pl.ds(s * rpt, rpt)],
                    out_hbm.at[c, pl.ds(s * rpt, rpt)])


def _edge_call(s_tab, d_tab, srcr, dstr, w_b, zeros_n2, n2, nblk):
    mesh = plsc.VectorSubcoreMesh(core_axis_name="c", subcore_axis_name="s")
    body = functools.partial(_edge_body, n2, nblk)
    return pl.kernel(
        body,
        out_type=jax.ShapeDtypeStruct((NCORE, n2, 16), F32),
        mesh=mesh,
        compiler_params=pltpu.CompilerParams(needs_layout_passes=False,
                                             use_tc_tiling_on_sc=False),
        scratch_types=[
            pltpu.VMEM_SHARED((n2, 16), F32),
            pltpu.VMEM((96, 16), F32),
            pltpu.VMEM((RPB, 128), jnp.int32),
            pltpu.VMEM((RPB, 128), jnp.int32),
            pltpu.VMEM((KB, 16), F32),
            pltpu.VMEM((KB, 16), F32),
            pltpu.VMEM((KB, 16), F32),
            pltpu.SemaphoreType.DMA,
            pltpu.SemaphoreType.DMA,
        ],
    )(s_tab, d_tab, srcr, dstr, w_b, zeros_n2)


def _node_body(layer, acc_ref, c0_ref, s_ref, d_ref, nl_ref,
               wnf_ref, wrws_ref, bws_ref, ws2_ref, bs2_ref, wrwp_ref,
               bwp_ref, sn_ref, dn_ref):
    a = acc_ref[0] + acc_ref[1]
    bn = a.shape[0]
    nf1 = a[:, 0:2]
    inv = 1.0 / jnp.maximum(a[:, 8:9], 1.0)
    nf2 = a[:, 2:8] * inv
    cp = c0_ref[...]
    for t in range(2):
        cp = cp + nf1[:, t:t + 1] * wnf_ref[t:t + 1, :]
    for t in range(6):
        cp = cp + nf2[:, t:t + 1] * wnf_ref[2 + t:3 + t, :]
    hc = _leaky(cp)
    hs = jnp.zeros((bn, 4), F32) + bws_ref[...]
    for t in range(8):
        hs = hs + hc[:, t:t + 1] * wrws_ref[t:t + 1, :]
    hs = _leaky(hs)
    res = jnp.zeros((bn, 2), F32) + bs2_ref[...]
    for t in range(4):
        res = res + hs[:, t:t + 1] * ws2_ref[t:t + 1, :]
    sc = 0.01 / float(layer ** 10)
    f0 = 0.95 + 0.1 / (1.0 + jnp.exp(-(res[:, 0:1] * sc)))
    f1 = 0.95 + 0.1 / (1.0 + jnp.exp(-(res[:, 1:2] * sc)))
    psum = a[:, 9:11]
    pfsum = a[:, 11:15]
    li2 = lax.broadcasted_iota(jnp.int32, (bn, 2), 1)
    li4 = lax.broadcasted_iota(jnp.int32, (bn, 4), 1)
    npwr = jnp.where(li2 == 0, psum * f0, psum)
    npwrf = jnp.where(li4 == 2, pfsum * f1, pfsum)
    nmask = nl_ref[...] == float(layer)
    s_old = s_ref[...]
    pwr_n = jnp.where(nmask, npwr, s_old[:, 8:10])
    pwrf_n = jnp.where(nmask, npwrf, s_old[:, 10:14])
    anew = jnp.zeros((bn, 8), F32) + bwp_ref[...]
    for t in range(8):
        anew = anew + hc[:, t:t + 1] * wrwp_ref[t:t + 1, :]
    a_n = jnp.where(nmask, anew, s_old[:, 0:8])
    z2 = jnp.zeros((bn, 2), F32)
    sn_ref[...] = jnp.concatenate([a_n, pwr_n, pwrf_n, z2], axis=1)
    em_n = jnp.where(nl_ref[...] == float(layer + 1), 1.0, 0.0).astype(F32)
    dn_ref[...] = jnp.concatenate([d_ref[...][:, 0:8], em_n, z2, z2, z2,
                                   jnp.zeros((bn, 1), F32)], axis=1)


def _node_call(layer, acc, c0, s_tab, d_tab, nl_p,
               wnf, wrws, bwsr, ws2p, bs2r, wrwp, bwpr, n2, bn):
    grid = (n2 // bn,)
    io = lambda i: (i, 0)
    i3 = lambda i: (0, i, 0)
    w0 = lambda i: (0, 0)
    return pl.pallas_call(
        functools.partial(_node_body, layer),
        grid=grid,
        in_specs=[
            pl.BlockSpec((NCORE, bn, 16), i3),
            pl.BlockSpec((bn, 8), io),
            pl.BlockSpec((bn, 16), io),
            pl.BlockSpec((bn, 16), io),
            pl.BlockSpec((bn, 1), io),
            pl.BlockSpec((8, 8), w0),
            pl.BlockSpec((8, 4), w0),
            pl.BlockSpec((1, 4), w0),
            pl.BlockSpec((8, 2), w0),
            pl.BlockSpec((1, 2), w0),
            pl.BlockSpec((8, 8), w0),
            pl.BlockSpec((1, 8), w0),
        ],
        out_specs=[pl.BlockSpec((bn, 16), io), pl.BlockSpec((bn, 16), io)],
        out_shape=[jax.ShapeDtypeStruct((n2, 16), F32),
                   jax.ShapeDtypeStruct((n2, 16), F32)],
    )(acc, c0, s_tab, d_tab, nl_p,
      wnf, wrws, bwsr, ws2p, bs2r, wrwp, bwpr)


def kernel(nf, pwr, pwr_feat, edge_index, node_level, Wp1, bp1, Wp2, bp2,
           Wr1, br1, Wr2, br2, Ws1, bs1, Ws2, bs2):
    N = nf.shape[0]
    E = edge_index.shape[1]
    n2 = ((N + 1 + 127) // 128) * 128
    e2 = ((E + NW * KB - 1) // (NW * KB)) * (NW * KB)
    nblk = e2 // (NW * KB)
    bn = 1264 if n2 % 1264 == 0 else 8

    src = edge_index[0]
    dst = edge_index[1]
    if e2 != E:
        src = jnp.concatenate([src, jnp.zeros((e2 - E,), jnp.int32)])
        dst = jnp.concatenate([dst, jnp.full((e2 - E,), N, jnp.int32)])
    srcr = src.reshape(e2 // 128, 128)
    dstr = dst.reshape(e2 // 128, 128)

    pad = [(0, n2 - N)]
    nf_p = jnp.pad(nf, pad + [(0, 0)])
    pwr_p = jnp.pad(pwr, pad + [(0, 0)])
    pwrf_p = jnp.pad(pwr_feat, pad + [(0, 0)])
    nl_p = jnp.pad(node_level.astype(F32)[:, None], pad + [(0, 0)],
                   constant_values=-1.0)

    wpt, wpb = Wp1[:128], Wp1[128:]
    wrt = Wr1[:128]
    wnf = Wr1[128:136]
    wrws = Wr2 @ Ws1
    bws = br2 @ Ws1 + bs1
    wrwp = Wr2 @ wpt
    bwp = br2 @ wpt
    ws2p = jnp.concatenate([Ws2, jnp.zeros((4, 2), F32)], axis=0)
    w_flat = jnp.concatenate([Wp2.reshape(-1), bp2,
                              jnp.zeros((96 - 81,), F32)])
    w_b = jnp.broadcast_to(w_flat[:, None], (96, 16)).astype(F32)
    zeros_n2 = jnp.zeros((n2, 16), F32)

    s_tab, d_tab, c0 = _init_call(nf_p, wpt, wpb, bp1[None, :], wrt,
                                  br1[None, :], nl_p, pwr_p, pwrf_p, n2, bn)
    for layer in (1, 2, 3):
        acc = _edge_call(s_tab, d_tab, srcr, dstr, w_b, zeros_n2, n2, nblk)
        s_tab, d_tab = _node_call(layer, acc, c0, s_tab, d_tab, nl_p,
                                  wnf, wrws, bws[None, :], ws2p,
                                  bs2[None, :], wrwp, bwp[None, :], n2, bn)
    return s_tab[:N, 8:10], s_tab[:N, 10:14]


# speedup vs baseline: 1.5622x; 1.5622x over previous
import functools

import jax
import jax.numpy as jnp
from jax import lax
from jax.experimental import pallas as pl
from jax.experimental.pallas import tpu as pltpu
from jax.experimental.pallas import tpu_sc as plsc

F32 = jnp.float32
NSUB = 16
NCORE = 2
NW = NCORE * NSUB
KB = 1024
RPB = KB // 128


def _leaky(x):
    return jnp.maximum(x, 0.2 * x)


def _init_body(nf_ref, wpt_ref, wpb_ref, bp1_ref, wrt_ref, br1_ref, nl_ref,
               pwr_ref, pwrf_ref, s_ref, d_ref, c0_ref):
    x = nf_ref[...]
    bn = x.shape[0]
    z2 = jnp.zeros((bn, 2), F32)
    a0 = jnp.dot(x, wpt_ref[...], preferred_element_type=F32)
    b0 = jnp.dot(x, wpb_ref[...], preferred_element_type=F32) + bp1_ref[...]
    c0_ref[...] = jnp.dot(x, wrt_ref[...], preferred_element_type=F32) + br1_ref[...]
    em1 = jnp.where(nl_ref[...] == 1.0, 1.0, 0.0).astype(F32)
    s_ref[...] = jnp.concatenate([a0, pwr_ref[...], pwrf_ref[...], z2], axis=1)
    d_ref[...] = jnp.concatenate([b0, em1, z2, z2, z2, jnp.zeros((bn, 1), F32)],
                                 axis=1)


def _init_call(nf_p, wpt, wpb, bp1r, wrt, br1r, nl_p, pwr_p, pwrf_p, n2, bn):
    grid = (n2 // bn,)
    io = lambda i: (i, 0)
    w0 = lambda i: (0, 0)
    return pl.pallas_call(
        _init_body,
        grid=grid,
        in_specs=[
            pl.BlockSpec((bn, 128), io),
            pl.BlockSpec((128, 8), w0),
            pl.BlockSpec((128, 8), w0),
            pl.BlockSpec((1, 8), w0),
            pl.BlockSpec((128, 8), w0),
            pl.BlockSpec((1, 8), w0),
            pl.BlockSpec((bn, 1), io),
            pl.BlockSpec((bn, 2), io),
            pl.BlockSpec((bn, 4), io),
        ],
        out_specs=[
            pl.BlockSpec((bn, 16), io),
            pl.BlockSpec((bn, 16), io),
            pl.BlockSpec((bn, 8), io),
        ],
        out_shape=[
            jax.ShapeDtypeStruct((n2, 16), F32),
            jax.ShapeDtypeStruct((n2, 16), F32),
            jax.ShapeDtypeStruct((n2, 8), F32),
        ],
    )(nf_p, wpt, wpb, bp1r, wrt, br1r, nl_p, pwr_p, pwrf_p)


def _edge_body(n2, nblk, s_hbm, d_hbm, srcr, dstr, w_hbm, z_hbm, out_hbm,
               acc, wbuf, idx_s, idx_d, sbuf, dbuf, obuf,
               idx_s2, idx_d2, sbuf2, dbuf2, obuf2, sem1, sem2):
    c = lax.axis_index("c")
    s = lax.axis_index("s")
    wid = c * NSUB + s
    rpt = n2 // NSUB
    pltpu.sync_copy(z_hbm.at[pl.ds(s * rpt, rpt)], acc.at[pl.ds(s * rpt, rpt)])
    pltpu.sync_copy(w_hbm, wbuf)
    plsc.subcore_barrier()

    row_base = wid * nblk * RPB
    iota = lax.iota(jnp.int32, 16)
    cidx = [jnp.full((16,), cc, jnp.int32) for cc in range(16)]

    def make_group_body(sb, db, ob):
        def group_body(g, car):
            rows = g * 16 + iota
            scol = [plsc.load_gather(sb, [rows, cidx[cc]]) for cc in range(14)]
            dcol = [plsc.load_gather(db, [rows, cidx[cc]]) for cc in range(9)]
            h = [_leaky(scol[f] + dcol[f]) for f in range(8)]
            e = []
            for j in range(9):
                acc_v = wbuf[72 + j]
                for f in range(8):
                    acc_v = acc_v + h[f] * wbuf[f * 9 + j]
                e.append(acc_v)
            m = dcol[8]
            km = m / (1.0 + jnp.exp(-e[0]))
            outc = [km * e[jj] for jj in range(1, 9)]
            outc.append(m)
            outc += [m * scol[8 + t] for t in range(6)]
            outc.append(jnp.zeros((16,), F32))
            for cc in range(16):
                plsc.store_scatter(ob, [rows, cidx[cc]], outc[cc])
            return car
        return group_body

    def fire(blk, idxs, idxd, sb, db, sem):
        r0 = row_base + blk * RPB
        pltpu.sync_copy(srcr.at[pl.ds(r0, RPB)], idxs)
        pltpu.sync_copy(dstr.at[pl.ds(r0, RPB)], idxd)
        for q in range(RPB):
            pltpu.async_copy(s_hbm.at[idxs.at[q]],
                             sb.at[pl.ds(q * 128, 128)], sem)
            pltpu.async_copy(d_hbm.at[idxd.at[q]],
                             db.at[pl.ds(q * 128, 128)], sem)

    def drain(idxs, idxd, sb, db, sem):
        for q in range(RPB):
            pltpu.make_async_copy(s_hbm.at[idxs.at[q]],
                                  sb.at[pl.ds(q * 128, 128)], sem).wait()
            pltpu.make_async_copy(d_hbm.at[idxd.at[q]],
                                  db.at[pl.ds(q * 128, 128)], sem).wait()

    def compute_and_scatter(idxd, sb, db, ob):
        lax.fori_loop(0, KB // 16, make_group_body(sb, db, ob), jnp.int32(0))
        for q in range(RPB):
            pltpu.sync_copy(ob.at[pl.ds(q * 128, 128)],
                            acc.at[idxd.at[q]], add=True)

    fire(0, idx_s, idx_d, sbuf, dbuf, sem1)

    def step(i, car):
        b0 = 2 * i
        fire(b0 + 1, idx_s2, idx_d2, sbuf2, dbuf2, sem2)
        drain(idx_s, idx_d, sbuf, dbuf, sem1)
        compute_and_scatter(idx_d, sbuf, dbuf, obuf)

        @pl.when(b0 + 2 < nblk)
        def _():
            fire(b0 + 2, idx_s, idx_d, sbuf, dbuf, sem1)
        drain(idx_s2, idx_d2, sbuf2, dbuf2, sem2)
        compute_and_scatter(idx_d2, sbuf2, dbuf2, obuf2)
        return car

    lax.fori_loop(0, nblk // 2, step, jnp.int32(0))
    if nblk % 2:
        drain(idx_s, idx_d, sbuf, dbuf, sem1)
        compute_and_scatter(idx_d, sbuf, dbuf, obuf)

    plsc.subcore_barrier()
    pltpu.sync_copy(acc.at[pl.ds(s * rpt, rpt)],
                    out_hbm.at[c, pl.ds(s * rpt, rpt)])


def _edge_call(s_tab, d_tab, srcr, dstr, w_b, zeros_n2, n2, nblk):
    mesh = plsc.VectorSubcoreMesh(core_axis_name="c", subcore_axis_name="s")
    body = functools.partial(_edge_body, n2, nblk)
    return pl.kernel(
        body,
        out_type=jax.ShapeDtypeStruct((NCORE, n2, 16), F32),
        mesh=mesh,
        compiler_params=pltpu.CompilerParams(needs_layout_passes=False,
                                             use_tc_tiling_on_sc=False),
        scratch_types=[
            pltpu.VMEM_SHARED((n2, 16), F32),
            pltpu.VMEM((96, 16), F32),
            pltpu.VMEM((RPB, 128), jnp.int32),
            pltpu.VMEM((RPB, 128), jnp.int32),
            pltpu.VMEM((KB, 16), F32),
            pltpu.VMEM((KB, 16), F32),
            pltpu.VMEM((KB, 16), F32),
            pltpu.VMEM((RPB, 128), jnp.int32),
            pltpu.VMEM((RPB, 128), jnp.int32),
            pltpu.VMEM((KB, 16), F32),
            pltpu.VMEM((KB, 16), F32),
            pltpu.VMEM((KB, 16), F32),
            pltpu.SemaphoreType.DMA,
            pltpu.SemaphoreType.DMA,
        ],
    )(s_tab, d_tab, srcr, dstr, w_b, zeros_n2)


def _node_body(layer, acc_ref, c0_ref, s_ref, d_ref, nl_ref,
               wnf_ref, wrws_ref, bws_ref, ws2_ref, bs2_ref, wrwp_ref,
               bwp_ref, sn_ref, dn_ref):
    a = acc_ref[0] + acc_ref[1]
    bn = a.shape[0]
    nf1 = a[:, 0:2]
    inv = 1.0 / jnp.maximum(a[:, 8:9], 1.0)
    nf2 = a[:, 2:8] * inv
    cp = c0_ref[...]
    for t in range(2):
        cp = cp + nf1[:, t:t + 1] * wnf_ref[t:t + 1, :]
    for t in range(6):
        cp = cp + nf2[:, t:t + 1] * wnf_ref[2 + t:3 + t, :]
    hc = _leaky(cp)
    hs = jnp.zeros((bn, 4), F32) + bws_ref[...]
    for t in range(8):
        hs = hs + hc[:, t:t + 1] * wrws_ref[t:t + 1, :]
    hs = _leaky(hs)
    res = jnp.zeros((bn, 2), F32) + bs2_ref[...]
    for t in range(4):
        res = res + hs[:, t:t + 1] * ws2_ref[t:t + 1, :]
    sc = 0.01 / float(layer ** 10)
    f0 = 0.95 + 0.1 / (1.0 + jnp.exp(-(res[:, 0:1] * sc)))
    f1 = 0.95 + 0.1 / (1.0 + jnp.exp(-(res[:, 1:2] * sc)))
    psum = a[:, 9:11]
    pfsum = a[:, 11:15]
    li2 = lax.broadcasted_iota(jnp.int32, (bn, 2), 1)
    li4 = lax.broadcasted_iota(jnp.int32, (bn, 4), 1)
    npwr = jnp.where(li2 == 0, psum * f0, psum)
    npwrf = jnp.where(li4 == 2, pfsum * f1, pfsum)
    nmask = nl_ref[...] == float(layer)
    s_old = s_ref[...]
    pwr_n = jnp.where(nmask, npwr, s_old[:, 8:10])
    pwrf_n = jnp.where(nmask, npwrf, s_old[:, 10:14])
    anew = jnp.zeros((bn, 8), F32) + bwp_ref[...]
    for t in range(8):
        anew = anew + hc[:, t:t + 1] * wrwp_ref[t:t + 1, :]
    a_n = jnp.where(nmask, anew, s_old[:, 0:8])
    z2 = jnp.zeros((bn, 2), F32)
    sn_ref[...] = jnp.concatenate([a_n, pwr_n, pwrf_n, z2], axis=1)
    em_n = jnp.where(nl_ref[...] == float(layer + 1), 1.0, 0.0).astype(F32)
    dn_ref[...] = jnp.concatenate([d_ref[...][:, 0:8], em_n, z2, z2, z2,
                                   jnp.zeros((bn, 1), F32)], axis=1)


def _node_call(layer, acc, c0, s_tab, d_tab, nl_p,
               wnf, wrws, bwsr, ws2p, bs2r, wrwp, bwpr, n2, bn):
    grid = (n2 // bn,)
    io = lambda i: (i, 0)
    i3 = lambda i: (0, i, 0)
    w0 = lambda i: (0, 0)
    return pl.pallas_call(
        functools.partial(_node_body, layer),
        grid=grid,
        in_specs=[
            pl.BlockSpec((NCORE, bn, 16), i3),
            pl.BlockSpec((bn, 8), io),
            pl.BlockSpec((bn, 16), io),
            pl.BlockSpec((bn, 16), io),
            pl.BlockSpec((bn, 1), io),
            pl.BlockSpec((8, 8), w0),
            pl.BlockSpec((8, 4), w0),
            pl.BlockSpec((1, 4), w0),
            pl.BlockSpec((8, 2), w0),
            pl.BlockSpec((1, 2), w0),
            pl.BlockSpec((8, 8), w0),
            pl.BlockSpec((1, 8), w0),
        ],
        out_specs=[pl.BlockSpec((bn, 16), io), pl.BlockSpec((bn, 16), io)],
        out_shape=[jax.ShapeDtypeStruct((n2, 16), F32),
                   jax.ShapeDtypeStruct((n2, 16), F32)],
    )(acc, c0, s_tab, d_tab, nl_p,
      wnf, wrws, bwsr, ws2p, bs2r, wrwp, bwpr)


def kernel(nf, pwr, pwr_feat, edge_index, node_level, Wp1, bp1, Wp2, bp2,
           Wr1, br1, Wr2, br2, Ws1, bs1, Ws2, bs2):
    N = nf.shape[0]
    E = edge_index.shape[1]
    n2 = ((N + 1 + 127) // 128) * 128
    e2 = ((E + NW * KB - 1) // (NW * KB)) * (NW * KB)
    nblk = e2 // (NW * KB)
    bn = 1264 if n2 % 1264 == 0 else 8

    src = edge_index[0]
    dst = edge_index[1]
    if e2 != E:
        src = jnp.concatenate([src, jnp.zeros((e2 - E,), jnp.int32)])
        dst = jnp.concatenate([dst, jnp.full((e2 - E,), N, jnp.int32)])
    srcr = src.reshape(e2 // 128, 128)
    dstr = dst.reshape(e2 // 128, 128)

    pad = [(0, n2 - N)]
    nf_p = jnp.pad(nf, pad + [(0, 0)])
    pwr_p = jnp.pad(pwr, pad + [(0, 0)])
    pwrf_p = jnp.pad(pwr_feat, pad + [(0, 0)])
    nl_p = jnp.pad(node_level.astype(F32)[:, None], pad + [(0, 0)],
                   constant_values=-1.0)

    wpt, wpb = Wp1[:128], Wp1[128:]
    wrt = Wr1[:128]
    wnf = Wr1[128:136]
    wrws = Wr2 @ Ws1
    bws = br2 @ Ws1 + bs1
    wrwp = Wr2 @ wpt
    bwp = br2 @ wpt
    ws2p = jnp.concatenate([Ws2, jnp.zeros((4, 2), F32)], axis=0)
    w_flat = jnp.concatenate([Wp2.reshape(-1), bp2,
                              jnp.zeros((96 - 81,), F32)])
    w_b = jnp.broadcast_to(w_flat[:, None], (96, 16)).astype(F32)
    zeros_n2 = jnp.zeros((n2, 16), F32)

    s_tab, d_tab, c0 = _init_call(nf_p, wpt, wpb, bp1[None, :], wrt,
                                  br1[None, :], nl_p, pwr_p, pwrf_p, n2, bn)
    for layer in (1, 2, 3):
        acc = _edge_call(s_tab, d_tab, srcr, dstr, w_b, zeros_n2, n2, nblk)
        s_tab, d_tab = _node_call(layer, acc, c0, s_tab, d_tab, nl_p,
                                  wnf, wrws, bws[None, :], ws2p,
                                  bs2[None, :], wrwp, bwp[None, :], n2, bn)
    return s_tab[:N, 8:10], s_tab[:N, 10:14]


# speedup vs baseline: 1.6706x; 1.0694x over previous
import functools

import jax
import jax.numpy as jnp
from jax import lax
from jax.experimental import pallas as pl
from jax.experimental.pallas import tpu as pltpu
from jax.experimental.pallas import tpu_sc as plsc

F32 = jnp.float32
NSUB = 16
NCORE = 2
NW = NCORE * NSUB
KB = 1024
RPB = KB // 128


def _leaky(x):
    return jnp.maximum(x, 0.2 * x)


def _init_body(nf_ref, wpt_ref, wpb_ref, bp1_ref, wrt_ref, br1_ref,
               pwr_ref, pwrf_ref, s_ref, d_ref, c0_ref):
    x = nf_ref[...]
    bn = x.shape[0]
    z2 = jnp.zeros((bn, 2), F32)
    a0 = jnp.dot(x, wpt_ref[...], preferred_element_type=F32)
    d_ref[...] = jnp.dot(x, wpb_ref[...], preferred_element_type=F32) + bp1_ref[...]
    c0_ref[...] = jnp.dot(x, wrt_ref[...], preferred_element_type=F32) + br1_ref[...]
    s_ref[...] = jnp.concatenate([a0, pwr_ref[...], pwrf_ref[...], z2], axis=1)


def _init_call(nf_p, wpt, wpb, bp1r, wrt, br1r, pwr_p, pwrf_p, n2, bn):
    grid = (n2 // bn,)
    io = lambda i: (i, 0)
    w0 = lambda i: (0, 0)
    return pl.pallas_call(
        _init_body,
        grid=grid,
        in_specs=[
            pl.BlockSpec((bn, 128), io),
            pl.BlockSpec((128, 8), w0),
            pl.BlockSpec((128, 8), w0),
            pl.BlockSpec((1, 8), w0),
            pl.BlockSpec((128, 8), w0),
            pl.BlockSpec((1, 8), w0),
            pl.BlockSpec((bn, 2), io),
            pl.BlockSpec((bn, 4), io),
        ],
        out_specs=[
            pl.BlockSpec((bn, 16), io),
            pl.BlockSpec((bn, 8), io),
            pl.BlockSpec((bn, 8), io),
        ],
        out_shape=[
            jax.ShapeDtypeStruct((n2, 16), F32),
            jax.ShapeDtypeStruct((n2, 8), F32),
            jax.ShapeDtypeStruct((n2, 8), F32),
        ],
    )(nf_p, wpt, wpb, bp1r, wrt, br1r, pwr_p, pwrf_p)


def _edge_body(n2, nblk, layer, s_hbm, d_hbm, srcr, dstr, w_hbm, z_hbm,
               nl_hbm, out_hbm,
               acc, wbuf, nlbuf, idx_s, idx_d, sbuf, dbuf, obuf,
               idx_s2, idx_d2, sbuf2, dbuf2, obuf2, sem1, sem2):
    c = lax.axis_index("c")
    s = lax.axis_index("s")
    wid = c * NSUB + s
    rpt = n2 // NSUB
    pltpu.sync_copy(z_hbm.at[pl.ds(s * rpt, rpt)], acc.at[pl.ds(s * rpt, rpt)])
    pltpu.sync_copy(w_hbm, wbuf)
    pltpu.sync_copy(nl_hbm, nlbuf)
    plsc.subcore_barrier()

    row_base = wid * nblk * RPB
    iota = lax.iota(jnp.int32, 16)
    cidx = [jnp.full((16,), cc, jnp.int32) for cc in range(16)]

    def make_group_body(sb, db, ob, idxd):
        def group_body(g, car):
            rows = g * 16 + iota
            dvec = idxd[g >> 3, pl.ds((g & 7) * 16, 16)]
            lvl = plsc.load_gather(nlbuf, [dvec])
            m = jnp.where(lvl == layer, 1.0, 0.0).astype(F32)
            scol = [plsc.load_gather(sb, [rows, cidx[cc]]) for cc in range(14)]
            dcol = [plsc.load_gather(db, [rows, cidx[cc]]) for cc in range(8)]
            h = [_leaky(scol[f] + dcol[f]) for f in range(8)]
            e = []
            for j in range(9):
                acc_v = wbuf[72 + j]
                for f in range(8):
                    acc_v = acc_v + h[f] * wbuf[f * 9 + j]
                e.append(acc_v)
            km = m / (1.0 + jnp.exp(-e[0]))
            outc = [km * e[jj] for jj in range(1, 9)]
            outc.append(m)
            outc += [m * scol[8 + t] for t in range(6)]
            outc.append(jnp.zeros((16,), F32))
            for cc in range(16):
                plsc.store_scatter(ob, [rows, cidx[cc]], outc[cc])
            return car
        return group_body

    def fire(blk, idxs, idxd, sb, db, sem):
        r0 = row_base + blk * RPB
        pltpu.sync_copy(srcr.at[pl.ds(r0, RPB)], idxs)
        pltpu.sync_copy(dstr.at[pl.ds(r0, RPB)], idxd)
        for q in range(RPB):
            pltpu.async_copy(s_hbm.at[idxs.at[q]],
                             sb.at[pl.ds(q * 128, 128)], sem)
            pltpu.async_copy(d_hbm.at[idxd.at[q]],
                             db.at[pl.ds(q * 128, 128)], sem)

    def drain(idxs, idxd, sb, db, sem):
        for q in range(RPB):
            pltpu.make_async_copy(s_hbm.at[idxs.at[q]],
                                  sb.at[pl.ds(q * 128, 128)], sem).wait()
            pltpu.make_async_copy(d_hbm.at[idxd.at[q]],
                                  db.at[pl.ds(q * 128, 128)], sem).wait()

    def compute_and_scatter(idxd, sb, db, ob):
        lax.fori_loop(0, KB // 16, make_group_body(sb, db, ob, idxd),
                      jnp.int32(0))
        for q in range(RPB):
            pltpu.sync_copy(ob.at[pl.ds(q * 128, 128)],
                            acc.at[idxd.at[q]], add=True)

    fire(0, idx_s, idx_d, sbuf, dbuf, sem1)

    def step(i, car):
        b0 = 2 * i
        fire(b0 + 1, idx_s2, idx_d2, sbuf2, dbuf2, sem2)
        drain(idx_s, idx_d, sbuf, dbuf, sem1)
        compute_and_scatter(idx_d, sbuf, dbuf, obuf)

        @pl.when(b0 + 2 < nblk)
        def _():
            fire(b0 + 2, idx_s, idx_d, sbuf, dbuf, sem1)
        drain(idx_s2, idx_d2, sbuf2, dbuf2, sem2)
        compute_and_scatter(idx_d2, sbuf2, dbuf2, obuf2)
        return car

    lax.fori_loop(0, nblk // 2, step, jnp.int32(0))
    if nblk % 2:
        drain(idx_s, idx_d, sbuf, dbuf, sem1)
        compute_and_scatter(idx_d, sbuf, dbuf, obuf)

    plsc.subcore_barrier()
    pltpu.sync_copy(acc.at[pl.ds(s * rpt, rpt)],
                    out_hbm.at[c, pl.ds(s * rpt, rpt)])


def _edge_call(s_tab, d_tab, srcr, dstr, w_b, zeros_n2, nl_i, n2, nblk,
               layer):
    mesh = plsc.VectorSubcoreMesh(core_axis_name="c", subcore_axis_name="s")
    body = functools.partial(_edge_body, n2, nblk, layer)
    return pl.kernel(
        body,
        out_type=jax.ShapeDtypeStruct((NCORE, n2, 16), F32),
        mesh=mesh,
        compiler_params=pltpu.CompilerParams(needs_layout_passes=False,
                                             use_tc_tiling_on_sc=False),
        scratch_types=[
            pltpu.VMEM_SHARED((n2, 16), F32),
            pltpu.VMEM((96, 16), F32),
            pltpu.VMEM((n2,), jnp.int32),
            pltpu.VMEM((RPB, 128), jnp.int32),
            pltpu.VMEM((RPB, 128), jnp.int32),
            pltpu.VMEM((KB, 16), F32),
            pltpu.VMEM((KB, 8), F32),
            pltpu.VMEM((KB, 16), F32),
            pltpu.VMEM((RPB, 128), jnp.int32),
            pltpu.VMEM((RPB, 128), jnp.int32),
            pltpu.VMEM((KB, 16), F32),
            pltpu.VMEM((KB, 8), F32),
            pltpu.VMEM((KB, 16), F32),
            pltpu.SemaphoreType.DMA,
            pltpu.SemaphoreType.DMA,
        ],
    )(s_tab, d_tab, srcr, dstr, w_b, zeros_n2, nl_i)


def _node_body(layer, acc_ref, c0_ref, s_ref, nl_ref,
               wnf_ref, wrws_ref, bws_ref, ws2_ref, bs2_ref, wrwp_ref,
               bwp_ref, sn_ref):
    a = acc_ref[0] + acc_ref[1]
    bn = a.shape[0]
    nf1 = a[:, 0:2]
    inv = 1.0 / jnp.maximum(a[:, 8:9], 1.0)
    nf2 = a[:, 2:8] * inv
    cp = c0_ref[...]
    for t in range(2):
        cp = cp + nf1[:, t:t + 1] * wnf_ref[t:t + 1, :]
    for t in range(6):
        cp = cp + nf2[:, t:t + 1] * wnf_ref[2 + t:3 + t, :]
    hc = _leaky(cp)
    hs = jnp.zeros((bn, 4), F32) + bws_ref[...]
    for t in range(8):
        hs = hs + hc[:, t:t + 1] * wrws_ref[t:t + 1, :]
    hs = _leaky(hs)
    res = jnp.zeros((bn, 2), F32) + bs2_ref[...]
    for t in range(4):
        res = res + hs[:, t:t + 1] * ws2_ref[t:t + 1, :]
    sc = 0.01 / float(layer ** 10)
    f0 = 0.95 + 0.1 / (1.0 + jnp.exp(-(res[:, 0:1] * sc)))
    f1 = 0.95 + 0.1 / (1.0 + jnp.exp(-(res[:, 1:2] * sc)))
    psum = a[:, 9:11]
    pfsum = a[:, 11:15]
    li2 = lax.broadcasted_iota(jnp.int32, (bn, 2), 1)
    li4 = lax.broadcasted_iota(jnp.int32, (bn, 4), 1)
    npwr = jnp.where(li2 == 0, psum * f0, psum)
    npwrf = jnp.where(li4 == 2, pfsum * f1, pfsum)
    nmask = nl_ref[...] == float(layer)
    s_old = s_ref[...]
    pwr_n = jnp.where(nmask, npwr, s_old[:, 8:10])
    pwrf_n = jnp.where(nmask, npwrf, s_old[:, 10:14])
    anew = jnp.zeros((bn, 8), F32) + bwp_ref[...]
    for t in range(8):
        anew = anew + hc[:, t:t + 1] * wrwp_ref[t:t + 1, :]
    a_n = jnp.where(nmask, anew, s_old[:, 0:8])
    z2 = jnp.zeros((bn, 2), F32)
    sn_ref[...] = jnp.concatenate([a_n, pwr_n, pwrf_n, z2], axis=1)


def _node_call(layer, acc, c0, s_tab, nl_p,
               wnf, wrws, bwsr, ws2p, bs2r, wrwp, bwpr, n2, bn):
    grid = (n2 // bn,)
    io = lambda i: (i, 0)
    i3 = lambda i: (0, i, 0)
    w0 = lambda i: (0, 0)
    return pl.pallas_call(
        functools.partial(_node_body, layer),
        grid=grid,
        in_specs=[
            pl.BlockSpec((NCORE, bn, 16), i3),
            pl.BlockSpec((bn, 8), io),
            pl.BlockSpec((bn, 16), io),
            pl.BlockSpec((bn, 1), io),
            pl.BlockSpec((8, 8), w0),
            pl.BlockSpec((8, 4), w0),
            pl.BlockSpec((1, 4), w0),
            pl.BlockSpec((8, 2), w0),
            pl.BlockSpec((1, 2), w0),
            pl.BlockSpec((8, 8), w0),
            pl.BlockSpec((1, 8), w0),
        ],
        out_specs=[pl.BlockSpec((bn, 16), io)],
        out_shape=[jax.ShapeDtypeStruct((n2, 16), F32)],
    )(acc, c0, s_tab, nl_p,
      wnf, wrws, bwsr, ws2p, bs2r, wrwp, bwpr)


def kernel(nf, pwr, pwr_feat, edge_index, node_level, Wp1, bp1, Wp2, bp2,
           Wr1, br1, Wr2, br2, Ws1, bs1, Ws2, bs2):
    N = nf.shape[0]
    E = edge_index.shape[1]
    n2 = ((N + 1 + 127) // 128) * 128
    e2 = ((E + NW * KB - 1) // (NW * KB)) * (NW * KB)
    nblk = e2 // (NW * KB)
    bn = 1264 if n2 % 1264 == 0 else 8

    src = edge_index[0]
    dst = edge_index[1]
    if e2 != E:
        src = jnp.concatenate([src, jnp.zeros((e2 - E,), jnp.int32)])
        dst = jnp.concatenate([dst, jnp.full((e2 - E,), N, jnp.int32)])
    srcr = src.reshape(e2 // 128, 128)
    dstr = dst.reshape(e2 // 128, 128)

    pad = [(0, n2 - N)]
    nf_p = jnp.pad(nf, pad + [(0, 0)])
    pwr_p = jnp.pad(pwr, pad + [(0, 0)])
    pwrf_p = jnp.pad(pwr_feat, pad + [(0, 0)])
    nl_p = jnp.pad(node_level.astype(F32)[:, None], pad + [(0, 0)],
                   constant_values=-1.0)
    nl_i = jnp.pad(node_level.astype(jnp.int32), pad, constant_values=-1)

    wpt, wpb = Wp1[:128], Wp1[128:]
    wrt = Wr1[:128]
    wnf = Wr1[128:136]
    wrws = Wr2 @ Ws1
    bws = br2 @ Ws1 + bs1
    wrwp = Wr2 @ wpt
    bwp = br2 @ wpt
    ws2p = jnp.concatenate([Ws2, jnp.zeros((4, 2), F32)], axis=0)
    w_flat = jnp.concatenate([Wp2.reshape(-1), bp2,
                              jnp.zeros((96 - 81,), F32)])
    w_b = jnp.broadcast_to(w_flat[:, None], (96, 16)).astype(F32)
    zeros_n2 = jnp.zeros((n2, 16), F32)

    s_tab, d_tab, c0 = _init_call(nf_p, wpt, wpb, bp1[None, :], wrt,
                                  br1[None, :], pwr_p, pwrf_p, n2, bn)
    for layer in (1, 2, 3):
        acc = _edge_call(s_tab, d_tab, srcr, dstr, w_b, zeros_n2, nl_i,
                         n2, nblk, layer)
        (s_tab,) = _node_call(layer, acc, c0, s_tab, nl_p,
                              wnf, wrws, bws[None, :], ws2p,
                              bs2[None, :], wrwp, bwp[None, :], n2, bn)
    return s_tab[:N, 8:10], s_tab[:N, 10:14]
